# baseline (speedup 1.0000x reference)
import jax
import jax.numpy as jnp
from jax import lax
from jax.experimental import pallas as pl
from jax.experimental.pallas import tpu as pltpu
from jax.experimental.pallas import tpu_sc as plsc

NU = 4000
NI = 6000
NN = NU + NI
NE = 320000
D = 128
NLAYER = 3

NC = 2
NS = 16
CH = 80
NCK0 = 192
NCK1 = 64
EPAD = NS * (NCK0 + NCK1) * CH
NP = 10240
RPT = NP // NS


def _sc_layer_body(x_hbm, packed_hbm, w_hbm, part_hbm,
                   r0_v, r1_v, r2_v, r3_v,
                   pa_v, pb_v, wa_v, wb_v,
                   d0_v, d1_v, d2_v, d3_v, acc,
                   g0, g1, g2, g3, s0, s1, s2, s3, fsm):
    cid = lax.axis_index("c")
    sid = lax.axis_index("s")
    if True:
        rows = (r0_v, r1_v, r2_v, r3_v)
        didx = (d0_v, d1_v, d2_v, d3_v)
        gsem = (g0, g1, g2, g3)
        ssem = (s0, s1, s2, s3)
        nck = jnp.where(cid == 0, NCK0, NCK1)
        cbase = jnp.where(cid == 0, sid * NCK0, NS * NCK0 + sid * NCK1)

        def scale(i, pbuf, wbuf):
            def grp(g, carry):
                wvec = wbuf[i, pl.ds(g * 16, 16)]
                r0 = g * 16
                for lane in range(16):
                    wspl = jnp.full((16,), wvec[lane], jnp.float32)
                    for j in range(8):
                        rows[i][r0 + lane, pl.ds(16 * j, 16)] = (
                            rows[i][r0 + lane, pl.ds(16 * j, 16)] * wspl)
                return carry

            lax.fori_loop(0, CH // 16, grp, 0)

        def zero_row(r, carry):
            for j in range(8):
                r2_v[r, pl.ds(16 * j, 16)] = jnp.zeros((16,), jnp.float32)
            return carry

        lax.fori_loop(0, CH, zero_row, 0)
        abase = sid * RPT
        for k in range(RPT // CH):
            pltpu.sync_copy(r2_v, acc.at[pl.ds(abase + k * CH, CH)])
        pltpu.sync_copy(packed_hbm.at[pl.ds(cbase, 4)], pa_v)
        pltpu.sync_copy(w_hbm.at[pl.ds(cbase, 4)], wa_v)
        plsc.subcore_barrier()

        def subiter(c0, pbuf, wbuf, pnext, wnext):
            cf = jnp.minimum(c0 + 4, nck - 4)
            fp = pltpu.async_copy(packed_hbm.at[pl.ds(cbase + cf, 4)], pnext, fsm)
            fw = pltpu.async_copy(w_hbm.at[pl.ds(cbase + cf, 4)], wnext, fsm)
            gd = [pltpu.async_copy(x_hbm.at[pbuf.at[i, 0]], rows[i], gsem[i])
                  for i in range(4)]
            sd = []
            for i in range(4):
                gd[i].wait()
                for g in range(CH // 16):
                    didx[i][pl.ds(16 * g, 16)] = pbuf[i, 1, pl.ds(16 * g, 16)]
                scale(i, pbuf, wbuf)
                sd.append(pltpu.async_copy(rows[i], acc.at[didx[i]], ssem[i],
                                           add=True))
            for d in sd:
                d.wait()
            fp.wait()
            fw.wait()

        def body(s2, carry):
            c0 = 8 * s2
            subiter(c0, pa_v, wa_v, pb_v, wb_v)
            subiter(c0 + 4, pb_v, wb_v, pa_v, wa_v)
            return carry

        lax.fori_loop(0, nck // 8, body, 0)
        plsc.subcore_barrier()

        pltpu.sync_copy(acc.at[pl.ds(abase, RPT)],
                        part_hbm.at[pl.ds(cid * NP + abase, RPT)])


@jax.jit
def _sc_layer(x, packed, w):
    mesh = plsc.VectorSubcoreMesh(core_axis_name="c", subcore_axis_name="s")
    return pl.kernel(
        _sc_layer_body,
        out_type=jax.ShapeDtypeStruct((NC * NP, D), jnp.float32),
        mesh=mesh,
        scratch_types=(
            [pltpu.VMEM((CH, D), jnp.float32)] * 4
            + [pltpu.VMEM((4, 2, CH), jnp.int32)] * 2
            + [pltpu.VMEM((4, CH), jnp.float32)] * 2
            + [pltpu.VMEM((CH,), jnp.int32)] * 4
            + [pltpu.VMEM_SHARED((NP, D), jnp.float32)]
            + [pltpu.SemaphoreType.DMA] * 9
        ),
    )(x, packed, w)


def _mean_body(x0_ref, x1_ref, x2_ref, x3_ref, m_ref):
    m_ref[...] = (x0_ref[...] + x1_ref[...] + x2_ref[...] + x3_ref[...]) * 0.25


def _add_body(p0_ref, p1_ref, x_ref):
    x_ref[...] = p0_ref[...] + p1_ref[...]


@jax.jit
def _add2(p0, p1):
    return pl.pallas_call(
        _add_body,
        grid=(NP // _BLK2,),
        in_specs=[_row_spec2(), _row_spec2()],
        out_specs=_row_spec2(),
        out_shape=jax.ShapeDtypeStruct((NP, D), jnp.float32),
    )(p0, p1)


_BLK = 1280
_BLK2 = 1280


def _row_spec():
    return pl.BlockSpec((_BLK, D), lambda i: (i, 0))


def _row_spec2():
    return pl.BlockSpec((_BLK2, D), lambda i: (i, 0))


@jax.jit
def _mean4(x0, x1, x2, x3):
    return pl.pallas_call(
        _mean_body,
        grid=(NP // _BLK,),
        in_specs=[_row_spec()] * 4,
        out_specs=_row_spec(),
        out_shape=jax.ShapeDtypeStruct((NP, D), jnp.float32),
    )(x0, x1, x2, x3)


def kernel(user_emb, item_emb, edge_weight, edge_index):
    x0 = jnp.pad(jnp.concatenate([user_emb, item_emb], axis=0),
                 ((0, NP - NN), (0, 0)))
    pad = EPAD - NE
    src = jnp.pad(edge_index[1], (0, pad)).reshape(-1, CH)
    dst = jnp.pad(edge_index[0], (0, pad)).reshape(-1, CH)
    w = jnp.pad(edge_weight, (0, pad)).reshape(-1, CH)
    packed = jnp.stack([src, dst], axis=1)

    xs = [x0]
    for layer in range(NLAYER):
        part = _sc_layer(xs[-1], packed, w)
        xs.append(_add2(part[:NP], part[NP:]))
    mean = _mean4(*xs)
    return (mean[:NU], mean[NU:NN])


# speedup vs baseline: 1.0059x; 1.0059x over previous
import jax
import jax.numpy as jnp
from jax import lax
from jax.experimental import pallas as pl
from jax.experimental.pallas import tpu as pltpu
from jax.experimental.pallas import tpu_sc as plsc

NU = 4000
NI = 6000
NN = NU + NI
NE = 320000
D = 128
NLAYER = 3

NC = 2
NS = 16
CH = 80
NCK0 = 192
NCK1 = 64
EPAD = NS * (NCK0 + NCK1) * CH
NP = 10240
RPT = NP // NS


def _sc_layer_body(x_hbm, packed_hbm, w_hbm, part_hbm,
                   r0_v, r1_v, r2_v, r3_v,
                   pa_v, pb_v, wa_v, wb_v,
                   d0_v, d1_v, d2_v, d3_v, acc,
                   g0, g1, g2, g3, s0, s1, s2, s3, fsm):
    cid = lax.axis_index("c")
    sid = lax.axis_index("s")
    if True:
        rows = (r0_v, r1_v, r2_v, r3_v)
        didx = (d0_v, d1_v, d2_v, d3_v)
        gsem = (g0, g1, g2, g3)
        ssem = (s0, s1, s2, s3)
        nck = jnp.where(cid == 0, NCK0, NCK1)
        cbase = jnp.where(cid == 0, sid * NCK0, NS * NCK0 + sid * NCK1)

        def scale(i, pbuf, wbuf):
            def grp(g, carry):
                wvec = wbuf[i, pl.ds(g * 16, 16)]
                r0 = g * 16
                for lane in range(16):
                    wspl = jnp.full((16,), wvec[lane], jnp.float32)
                    for j in range(8):
                        rows[i][r0 + lane, pl.ds(16 * j, 16)] = (
                            rows[i][r0 + lane, pl.ds(16 * j, 16)] * wspl)
                return carry

            lax.fori_loop(0, CH // 16, grp, 0)

        def zero_row(r, carry):
            for j in range(8):
                r2_v[r, pl.ds(16 * j, 16)] = jnp.zeros((16,), jnp.float32)
            return carry

        lax.fori_loop(0, CH, zero_row, 0)
        abase = sid * RPT
        for k in range(RPT // CH):
            pltpu.sync_copy(r2_v, acc.at[pl.ds(abase + k * CH, CH)])
        pltpu.sync_copy(packed_hbm.at[pl.ds(cbase, 4)], pa_v)
        pltpu.sync_copy(w_hbm.at[pl.ds(cbase, 4)], wa_v)
        plsc.subcore_barrier()

        def subiter(c0, pbuf, wbuf, pnext, wnext):
            cf = jnp.minimum(c0 + 4, nck - 4)
            fp = pltpu.async_copy(packed_hbm.at[pl.ds(cbase + cf, 4)], pnext, fsm)
            fw = pltpu.async_copy(w_hbm.at[pl.ds(cbase + cf, 4)], wnext, fsm)
            gd = [pltpu.async_copy(x_hbm.at[pbuf.at[i, 0]], rows[i], gsem[i])
                  for i in range(4)]
            sd = []
            for i in range(4):
                gd[i].wait()
                for g in range(CH // 16):
                    didx[i][pl.ds(16 * g, 16)] = pbuf[i, 1, pl.ds(16 * g, 16)]
                scale(i, pbuf, wbuf)
                sd.append(pltpu.async_copy(rows[i], acc.at[didx[i]], ssem[i],
                                           add=True))
            for d in sd:
                d.wait()
            fp.wait()
            fw.wait()

        def body(s2, carry):
            c0 = 8 * s2
            subiter(c0, pa_v, wa_v, pb_v, wb_v)
            subiter(c0 + 4, pb_v, wb_v, pa_v, wa_v)
            return carry

        lax.fori_loop(0, nck // 8, body, 0)
        plsc.subcore_barrier()

        pltpu.sync_copy(acc.at[pl.ds(abase, RPT)],
                        part_hbm.at[pl.ds(cid * NP + abase, RPT)])


@jax.jit
def _sc_layer(x, packed, w):
    mesh = plsc.VectorSubcoreMesh(core_axis_name="c", subcore_axis_name="s")
    return pl.kernel(
        _sc_layer_body,
        out_type=jax.ShapeDtypeStruct((NC * NP, D), jnp.float32),
        mesh=mesh,
        scratch_types=(
            [pltpu.VMEM((CH, D), jnp.float32)] * 4
            + [pltpu.VMEM((4, 2, CH), jnp.int32)] * 2
            + [pltpu.VMEM((4, CH), jnp.float32)] * 2
            + [pltpu.VMEM((CH,), jnp.int32)] * 4
            + [pltpu.VMEM_SHARED((NP, D), jnp.float32)]
            + [pltpu.SemaphoreType.DMA] * 9
        ),
    )(x, packed, w)


def _mean_body(p0_ref, p1_ref, x0_ref, x1_ref, x2_ref, m_ref):
    m_ref[...] = (x0_ref[...] + x1_ref[...] + x2_ref[...]
                  + p0_ref[...] + p1_ref[...]) * 0.25


def _add_body(p0_ref, p1_ref, x_ref):
    x_ref[...] = p0_ref[...] + p1_ref[...]


@jax.jit
def _add2(p0, p1):
    return pl.pallas_call(
        _add_body,
        grid=(NP // _BLK2,),
        in_specs=[_row_spec2(), _row_spec2()],
        out_specs=_row_spec2(),
        out_shape=jax.ShapeDtypeStruct((NP, D), jnp.float32),
    )(p0, p1)


_BLK = 1280
_BLK2 = 1280


def _row_spec():
    return pl.BlockSpec((_BLK, D), lambda i: (i, 0))


def _row_spec2():
    return pl.BlockSpec((_BLK2, D), lambda i: (i, 0))


@jax.jit
def _mean5(p0, p1, x0, x1, x2):
    return pl.pallas_call(
        _mean_body,
        grid=(NP // _BLK,),
        in_specs=[_row_spec()] * 5,
        out_specs=_row_spec(),
        out_shape=jax.ShapeDtypeStruct((NP, D), jnp.float32),
    )(p0, p1, x0, x1, x2)


def kernel(user_emb, item_emb, edge_weight, edge_index):
    x0 = jnp.pad(jnp.concatenate([user_emb, item_emb], axis=0),
                 ((0, NP - NN), (0, 0)))
    pad = EPAD - NE
    src = jnp.pad(edge_index[1], (0, pad)).reshape(-1, CH)
    dst = jnp.pad(edge_index[0], (0, pad)).reshape(-1, CH)
    w = jnp.pad(edge_weight, (0, pad)).reshape(-1, CH)
    packed = jnp.stack([src, dst], axis=1)

    xs = [x0]
    for layer in range(NLAYER - 1):
        part = _sc_layer(xs[-1], packed, w)
        xs.append(_add2(part[:NP], part[NP:]))
    part = _sc_layer(xs[-1], packed, w)
    mean = _mean5(part[:NP], part[NP:], *xs)
    return (mean[:NU], mean[NU:NN])


# speedup vs baseline: 1.0814x; 1.0751x over previous
import jax
import jax.numpy as jnp
from jax import lax
from jax.experimental import pallas as pl
from jax.experimental.pallas import tpu as pltpu
from jax.experimental.pallas import tpu_sc as plsc

NU = 4000
NI = 6000
NN = NU + NI
NE = 320000
D = 128
NLAYER = 3

NC = 2
NS = 16
CH = 80
NCK0 = 216
NCK1 = 40
EPAD = NS * (NCK0 + NCK1) * CH
NP = 10240
RPT = NP // NS


def _sc_layer_body(x_hbm, packed_hbm, w_hbm, part_hbm,
                   r0_v, r1_v, r2_v, r3_v,
                   pa_v, pb_v, wa_v, wb_v,
                   d0_v, d1_v, d2_v, d3_v, acc,
                   g0, g1, g2, g3, s0, s1, s2, s3, fsm):
    cid = lax.axis_index("c")
    sid = lax.axis_index("s")
    if True:
        rows = (r0_v, r1_v, r2_v, r3_v)
        didx = (d0_v, d1_v, d2_v, d3_v)
        gsem = (g0, g1, g2, g3)
        ssem = (s0, s1, s2, s3)
        nck = jnp.where(cid == 0, NCK0, NCK1)
        cbase = jnp.where(cid == 0, sid * NCK0, NS * NCK0 + sid * NCK1)

        def scale(i, pbuf, wbuf):
            def grp(g, carry):
                wvec = wbuf[i, pl.ds(g * 16, 16)]
                r0 = g * 16
                for lane in range(16):
                    wspl = jnp.full((16,), wvec[lane], jnp.float32)
                    for j in range(8):
                        rows[i][r0 + lane, pl.ds(16 * j, 16)] = (
                            rows[i][r0 + lane, pl.ds(16 * j, 16)] * wspl)
                return carry

            lax.fori_loop(0, CH // 16, grp, 0)

        def zero_row(r, carry):
            for j in range(8):
                r2_v[r, pl.ds(16 * j, 16)] = jnp.zeros((16,), jnp.float32)
            return carry

        lax.fori_loop(0, CH, zero_row, 0)
        abase = sid * RPT
        for k in range(RPT // CH):
            pltpu.sync_copy(r2_v, acc.at[pl.ds(abase + k * CH, CH)])
        pltpu.sync_copy(packed_hbm.at[pl.ds(cbase, 4)], pa_v)
        pltpu.sync_copy(w_hbm.at[pl.ds(cbase, 4)], wa_v)
        plsc.subcore_barrier()

        def subiter(c0, pbuf, wbuf, pnext, wnext):
            cf = jnp.minimum(c0 + 4, nck - 4)
            fp = pltpu.async_copy(packed_hbm.at[pl.ds(cbase + cf, 4)], pnext, fsm)
            fw = pltpu.async_copy(w_hbm.at[pl.ds(cbase + cf, 4)], wnext, fsm)
            gd = [pltpu.async_copy(x_hbm.at[pbuf.at[i, 0]], rows[i], gsem[i])
                  for i in range(4)]
            sd = []
            for i in range(4):
                gd[i].wait()
                for g in range(CH // 16):
                    didx[i][pl.ds(16 * g, 16)] = pbuf[i, 1, pl.ds(16 * g, 16)]
                scale(i, pbuf, wbuf)
                sd.append(pltpu.async_copy(rows[i], acc.at[didx[i]], ssem[i],
                                           add=True))
            for d in sd:
                d.wait()
            fp.wait()
            fw.wait()

        def body(s2, carry):
            c0 = 8 * s2
            subiter(c0, pa_v, wa_v, pb_v, wb_v)
            subiter(c0 + 4, pb_v, wb_v, pa_v, wa_v)
            return carry

        lax.fori_loop(0, nck // 8, body, 0)
        plsc.subcore_barrier()

        pltpu.sync_copy(acc.at[pl.ds(abase, RPT)],
                        part_hbm.at[pl.ds(cid * NP + abase, RPT)])


@jax.jit
def _sc_layer(x, packed, w):
    mesh = plsc.VectorSubcoreMesh(core_axis_name="c", subcore_axis_name="s")
    return pl.kernel(
        _sc_layer_body,
        out_type=jax.ShapeDtypeStruct((NC * NP, D), jnp.float32),
        mesh=mesh,
        scratch_types=(
            [pltpu.VMEM((CH, D), jnp.float32)] * 4
            + [pltpu.VMEM((4, 2, CH), jnp.int32)] * 2
            + [pltpu.VMEM((4, CH), jnp.float32)] * 2
            + [pltpu.VMEM((CH,), jnp.int32)] * 4
            + [pltpu.VMEM_SHARED((NP, D), jnp.float32)]
            + [pltpu.SemaphoreType.DMA] * 9
        ),
    )(x, packed, w)


def _mean_body(p0_ref, p1_ref, x0_ref, x1_ref, x2_ref, m_ref):
    m_ref[...] = (x0_ref[...] + x1_ref[...] + x2_ref[...]
                  + p0_ref[...] + p1_ref[...]) * 0.25


def _add_body(p0_ref, p1_ref, x_ref):
    x_ref[...] = p0_ref[...] + p1_ref[...]


@jax.jit
def _add2(p0, p1):
    return pl.pallas_call(
        _add_body,
        grid=(NP // _BLK2,),
        in_specs=[_row_spec2(), _row_spec2()],
        out_specs=_row_spec2(),
        out_shape=jax.ShapeDtypeStruct((NP, D), jnp.float32),
    )(p0, p1)


_BLK = 1280
_BLK2 = 1280


def _row_spec():
    return pl.BlockSpec((_BLK, D), lambda i: (i, 0))


def _row_spec2():
    return pl.BlockSpec((_BLK2, D), lambda i: (i, 0))


@jax.jit
def _mean5(p0, p1, x0, x1, x2):
    return pl.pallas_call(
        _mean_body,
        grid=(NP // _BLK,),
        in_specs=[_row_spec()] * 5,
        out_specs=_row_spec(),
        out_shape=jax.ShapeDtypeStruct((NP, D), jnp.float32),
    )(p0, p1, x0, x1, x2)


def kernel(user_emb, item_emb, edge_weight, edge_index):
    x0 = jnp.pad(jnp.concatenate([user_emb, item_emb], axis=0),
                 ((0, NP - NN), (0, 0)))
    pad = EPAD - NE
    src = jnp.pad(edge_index[1], (0, pad)).reshape(-1, CH)
    dst = jnp.pad(edge_index[0], (0, pad)).reshape(-1, CH)
    w = jnp.pad(edge_weight, (0, pad)).reshape(-1, CH)
    packed = jnp.stack([src, dst], axis=1)

    xs = [x0]
    for layer in range(NLAYER - 1):
        part = _sc_layer(xs[-1], packed, w)
        xs.append(_add2(part[:NP], part[NP:]))
    part = _sc_layer(xs[-1], packed, w)
    mean = _mean5(part[:NP], part[NP:], *xs)
    return (mean[:NU], mean[NU:NN])


# speedup vs baseline: 1.1244x; 1.0398x over previous
import jax
import jax.numpy as jnp
from jax import lax
from jax.experimental import pallas as pl
from jax.experimental.pallas import tpu as pltpu
from jax.experimental.pallas import tpu_sc as plsc

NU = 4000
NI = 6000
NN = NU + NI
NE = 320000
D = 128
NLAYER = 3

NC = 2
NS = 16
CH = 80
NCK0 = 224
NCK1 = 32
EPAD = NS * (NCK0 + NCK1) * CH
NP = 10240
RPT = NP // NS


def _sc_layer_body(x_hbm, packed_hbm, w_hbm, part_hbm,
                   r0_v, r1_v, r2_v, r3_v,
                   pa_v, pb_v, wa_v, wb_v,
                   d0_v, d1_v, d2_v, d3_v, acc,
                   g0, g1, g2, g3, s0, s1, s2, s3, fsm):
    cid = lax.axis_index("c")
    sid = lax.axis_index("s")
    if True:
        rows = (r0_v, r1_v, r2_v, r3_v)
        didx = (d0_v, d1_v, d2_v, d3_v)
        gsem = (g0, g1, g2, g3)
        ssem = (s0, s1, s2, s3)
        nck = jnp.where(cid == 0, NCK0, NCK1)
        cbase = jnp.where(cid == 0, sid * NCK0, NS * NCK0 + sid * NCK1)

        def scale(i, pbuf, wbuf):
            def grp(g, carry):
                wvec = wbuf[i, pl.ds(g * 16, 16)]
                r0 = g * 16
                for lane in range(16):
                    wspl = jnp.full((16,), wvec[lane], jnp.float32)
                    for j in range(8):
                        rows[i][r0 + lane, pl.ds(16 * j, 16)] = (
                            rows[i][r0 + lane, pl.ds(16 * j, 16)] * wspl)
                return carry

            lax.fori_loop(0, CH // 16, grp, 0)

        def zero_row(r, carry):
            for j in range(8):
                r2_v[r, pl.ds(16 * j, 16)] = jnp.zeros((16,), jnp.float32)
            return carry

        lax.fori_loop(0, CH, zero_row, 0)
        abase = sid * RPT
        for k in range(RPT // CH):
            pltpu.sync_copy(r2_v, acc.at[pl.ds(abase + k * CH, CH)])
        pltpu.sync_copy(packed_hbm.at[pl.ds(cbase, 4)], pa_v)
        pltpu.sync_copy(w_hbm.at[pl.ds(cbase, 4)], wa_v)
        plsc.subcore_barrier()

        def subiter(c0, pbuf, wbuf, pnext, wnext):
            cf = jnp.minimum(c0 + 4, nck - 4)
            fp = pltpu.async_copy(packed_hbm.at[pl.ds(cbase + cf, 4)], pnext, fsm)
            fw = pltpu.async_copy(w_hbm.at[pl.ds(cbase + cf, 4)], wnext, fsm)
            gd = [pltpu.async_copy(x_hbm.at[pbuf.at[i, 0]], rows[i], gsem[i])
                  for i in range(4)]
            sd = []
            for i in range(4):
                gd[i].wait()
                for g in range(CH // 16):
                    didx[i][pl.ds(16 * g, 16)] = pbuf[i, 1, pl.ds(16 * g, 16)]
                scale(i, pbuf, wbuf)
                sd.append(pltpu.async_copy(rows[i], acc.at[didx[i]], ssem[i],
                                           add=True))
            for d in sd:
                d.wait()
            fp.wait()
            fw.wait()

        def body(s2, carry):
            c0 = 8 * s2
            subiter(c0, pa_v, wa_v, pb_v, wb_v)
            subiter(c0 + 4, pb_v, wb_v, pa_v, wa_v)
            return carry

        lax.fori_loop(0, nck // 8, body, 0)
        plsc.subcore_barrier()

        pltpu.sync_copy(acc.at[pl.ds(abase, RPT)],
                        part_hbm.at[pl.ds(cid * NP + abase, RPT)])


@jax.jit
def _sc_layer(x, packed, w):
    mesh = plsc.VectorSubcoreMesh(core_axis_name="c", subcore_axis_name="s")
    return pl.kernel(
        _sc_layer_body,
        out_type=jax.ShapeDtypeStruct((NC * NP, D), jnp.float32),
        mesh=mesh,
        scratch_types=(
            [pltpu.VMEM((CH, D), jnp.float32)] * 4
            + [pltpu.VMEM((4, 2, CH), jnp.int32)] * 2
            + [pltpu.VMEM((4, CH), jnp.float32)] * 2
            + [pltpu.VMEM((CH,), jnp.int32)] * 4
            + [pltpu.VMEM_SHARED((NP, D), jnp.float32)]
            + [pltpu.SemaphoreType.DMA] * 9
        ),
    )(x, packed, w)


def _mean_body(p0_ref, p1_ref, x0_ref, x1_ref, x2_ref, m_ref):
    m_ref[...] = (x0_ref[...] + x1_ref[...] + x2_ref[...]
                  + p0_ref[...] + p1_ref[...]) * 0.25


def _add_body(p0_ref, p1_ref, x_ref):
    x_ref[...] = p0_ref[...] + p1_ref[...]


@jax.jit
def _add2(p0, p1):
    return pl.pallas_call(
        _add_body,
        grid=(NP // _BLK2,),
        in_specs=[_row_spec2(), _row_spec2()],
        out_specs=_row_spec2(),
        out_shape=jax.ShapeDtypeStruct((NP, D), jnp.float32),
    )(p0, p1)


_BLK = 1280
_BLK2 = 1280


def _row_spec():
    return pl.BlockSpec((_BLK, D), lambda i: (i, 0))


def _row_spec2():
    return pl.BlockSpec((_BLK2, D), lambda i: (i, 0))


@jax.jit
def _mean5(p0, p1, x0, x1, x2):
    return pl.pallas_call(
        _mean_body,
        grid=(NP // _BLK,),
        in_specs=[_row_spec()] * 5,
        out_specs=_row_spec(),
        out_shape=jax.ShapeDtypeStruct((NP, D), jnp.float32),
    )(p0, p1, x0, x1, x2)


def kernel(user_emb, item_emb, edge_weight, edge_index):
    x0 = jnp.pad(jnp.concatenate([user_emb, item_emb], axis=0),
                 ((0, NP - NN), (0, 0)))
    pad = EPAD - NE
    src = jnp.pad(edge_index[1], (0, pad)).reshape(-1, CH)
    dst = jnp.pad(edge_index[0], (0, pad)).reshape(-1, CH)
    w = jnp.pad(edge_weight, (0, pad)).reshape(-1, CH)
    packed = jnp.stack([src, dst], axis=1)

    xs = [x0]
    for layer in range(NLAYER - 1):
        part = _sc_layer(xs[-1], packed, w)
        xs.append(_add2(part[:NP], part[NP:]))
    part = _sc_layer(xs[-1], packed, w)
    mean = _mean5(part[:NP], part[NP:], *xs)
    return (mean[:NU], mean[NU:NN])


# speedup vs baseline: 1.2353x; 1.0986x over previous
import jax
import jax.numpy as jnp
from jax import lax
from jax.experimental import pallas as pl
from jax.experimental.pallas import tpu as pltpu
from jax.experimental.pallas import tpu_sc as plsc

NU = 4000
NI = 6000
NN = NU + NI
NE = 320000
D = 128
NLAYER = 3

NC = 2
NS = 16
CH = 80
NCK0 = 232
NCK1 = 24
EPAD = NS * (NCK0 + NCK1) * CH
NP = 10240
RPT = NP // NS


def _sc_layer_body(x_hbm, packed_hbm, w_hbm, part_hbm,
                   r0_v, r1_v, r2_v, r3_v,
                   pa_v, pb_v, wa_v, wb_v,
                   d0_v, d1_v, d2_v, d3_v, acc,
                   g0, g1, g2, g3, s0, s1, s2, s3, fsm):
    cid = lax.axis_index("c")
    sid = lax.axis_index("s")
    if True:
        rows = (r0_v, r1_v, r2_v, r3_v)
        didx = (d0_v, d1_v, d2_v, d3_v)
        gsem = (g0, g1, g2, g3)
        ssem = (s0, s1, s2, s3)
        nck = jnp.where(cid == 0, NCK0, NCK1)
        cbase = jnp.where(cid == 0, sid * NCK0, NS * NCK0 + sid * NCK1)

        def scale(i, pbuf, wbuf):
            def grp(g, carry):
                wvec = wbuf[i, pl.ds(g * 16, 16)]
                r0 = g * 16
                for lane in range(16):
                    wspl = jnp.full((16,), wvec[lane], jnp.float32)
                    for j in range(8):
                        rows[i][r0 + lane, pl.ds(16 * j, 16)] = (
                            rows[i][r0 + lane, pl.ds(16 * j, 16)] * wspl)
                return carry

            lax.fori_loop(0, CH // 16, grp, 0)

        def zero_row(r, carry):
            for j in range(8):
                r2_v[r, pl.ds(16 * j, 16)] = jnp.zeros((16,), jnp.float32)
            return carry

        lax.fori_loop(0, CH, zero_row, 0)
        abase = sid * RPT
        for k in range(RPT // CH):
            pltpu.sync_copy(r2_v, acc.at[pl.ds(abase + k * CH, CH)])
        pltpu.sync_copy(packed_hbm.at[pl.ds(cbase, 4)], pa_v)
        pltpu.sync_copy(w_hbm.at[pl.ds(cbase, 4)], wa_v)
        plsc.subcore_barrier()

        def subiter(c0, pbuf, wbuf, pnext, wnext):
            cf = jnp.minimum(c0 + 4, nck - 4)
            fp = pltpu.async_copy(packed_hbm.at[pl.ds(cbase + cf, 4)], pnext, fsm)
            fw = pltpu.async_copy(w_hbm.at[pl.ds(cbase + cf, 4)], wnext, fsm)
            gd = [pltpu.async_copy(x_hbm.at[pbuf.at[i, 0]], rows[i], gsem[i])
                  for i in range(4)]
            sd = []
            for i in range(4):
                gd[i].wait()
                for g in range(CH // 16):
                    didx[i][pl.ds(16 * g, 16)] = pbuf[i, 1, pl.ds(16 * g, 16)]
                scale(i, pbuf, wbuf)
                sd.append(pltpu.async_copy(rows[i], acc.at[didx[i]], ssem[i],
                                           add=True))
            for d in sd:
                d.wait()
            fp.wait()
            fw.wait()

        def body(s2, carry):
            c0 = 8 * s2
            subiter(c0, pa_v, wa_v, pb_v, wb_v)
            subiter(c0 + 4, pb_v, wb_v, pa_v, wa_v)
            return carry

        lax.fori_loop(0, nck // 8, body, 0)
        plsc.subcore_barrier()

        pltpu.sync_copy(acc.at[pl.ds(abase, RPT)],
                        part_hbm.at[pl.ds(cid * NP + abase, RPT)])


@jax.jit
def _sc_layer(x, packed, w):
    mesh = plsc.VectorSubcoreMesh(core_axis_name="c", subcore_axis_name="s")
    return pl.kernel(
        _sc_layer_body,
        out_type=jax.ShapeDtypeStruct((NC * NP, D), jnp.float32),
        mesh=mesh,
        scratch_types=(
            [pltpu.VMEM((CH, D), jnp.float32)] * 4
            + [pltpu.VMEM((4, 2, CH), jnp.int32)] * 2
            + [pltpu.VMEM((4, CH), jnp.float32)] * 2
            + [pltpu.VMEM((CH,), jnp.int32)] * 4
            + [pltpu.VMEM_SHARED((NP, D), jnp.float32)]
            + [pltpu.SemaphoreType.DMA] * 9
        ),
    )(x, packed, w)


def _mean_body(p0_ref, p1_ref, x0_ref, x1_ref, x2_ref, m_ref):
    m_ref[...] = (x0_ref[...] + x1_ref[...] + x2_ref[...]
                  + p0_ref[...] + p1_ref[...]) * 0.25


def _add_body(p0_ref, p1_ref, x_ref):
    x_ref[...] = p0_ref[...] + p1_ref[...]


@jax.jit
def _add2(p0, p1):
    return pl.pallas_call(
        _add_body,
        grid=(NP // _BLK2,),
        in_specs=[_row_spec2(), _row_spec2()],
        out_specs=_row_spec2(),
        out_shape=jax.ShapeDtypeStruct((NP, D), jnp.float32),
    )(p0, p1)


_BLK = 1280
_BLK2 = 1280


def _row_spec():
    return pl.BlockSpec((_BLK, D), lambda i: (i, 0))


def _row_spec2():
    return pl.BlockSpec((_BLK2, D), lambda i: (i, 0))


@jax.jit
def _mean5(p0, p1, x0, x1, x2):
    return pl.pallas_call(
        _mean_body,
        grid=(NP // _BLK,),
        in_specs=[_row_spec()] * 5,
        out_specs=_row_spec(),
        out_shape=jax.ShapeDtypeStruct((NP, D), jnp.float32),
    )(p0, p1, x0, x1, x2)


def kernel(user_emb, item_emb, edge_weight, edge_index):
    x0 = jnp.pad(jnp.concatenate([user_emb, item_emb], axis=0),
                 ((0, NP - NN), (0, 0)))
    pad = EPAD - NE
    src = jnp.pad(edge_index[1], (0, pad)).reshape(-1, CH)
    dst = jnp.pad(edge_index[0], (0, pad)).reshape(-1, CH)
    w = jnp.pad(edge_weight, (0, pad)).reshape(-1, CH)
    packed = jnp.stack([src, dst], axis=1)

    xs = [x0]
    for layer in range(NLAYER - 1):
        part = _sc_layer(xs[-1], packed, w)
        xs.append(_add2(part[:NP], part[NP:]))
    part = _sc_layer(xs[-1], packed, w)
    mean = _mean5(part[:NP], part[NP:], *xs)
    return (mean[:NU], mean[NU:NN])


# speedup vs baseline: 1.2359x; 1.0005x over previous
import jax
import jax.numpy as jnp
from jax import lax
from jax.experimental import pallas as pl
from jax.experimental.pallas import tpu as pltpu
from jax.experimental.pallas import tpu_sc as plsc

NU = 4000
NI = 6000
NN = NU + NI
NE = 320000
D = 128
NLAYER = 3

NC = 2
NS = 16
CH = 80
NCK0 = 240
NCK1 = 16
EPAD = NS * (NCK0 + NCK1) * CH
NP = 10240
RPT = NP // NS


def _sc_layer_body(x_hbm, packed_hbm, w_hbm, part_hbm,
                   r0_v, r1_v, r2_v, r3_v,
                   pa_v, pb_v, wa_v, wb_v,
                   d0_v, d1_v, d2_v, d3_v, acc,
                   g0, g1, g2, g3, s0, s1, s2, s3, fsm):
    cid = lax.axis_index("c")
    sid = lax.axis_index("s")
    if True:
        rows = (r0_v, r1_v, r2_v, r3_v)
        didx = (d0_v, d1_v, d2_v, d3_v)
        gsem = (g0, g1, g2, g3)
        ssem = (s0, s1, s2, s3)
        nck = jnp.where(cid == 0, NCK0, NCK1)
        cbase = jnp.where(cid == 0, sid * NCK0, NS * NCK0 + sid * NCK1)

        def scale(i, pbuf, wbuf):
            def grp(g, carry):
                wvec = wbuf[i, pl.ds(g * 16, 16)]
                r0 = g * 16
                for lane in range(16):
                    wspl = jnp.full((16,), wvec[lane], jnp.float32)
                    for j in range(8):
                        rows[i][r0 + lane, pl.ds(16 * j, 16)] = (
                            rows[i][r0 + lane, pl.ds(16 * j, 16)] * wspl)
                return carry

            lax.fori_loop(0, CH // 16, grp, 0)

        def zero_row(r, carry):
            for j in range(8):
                r2_v[r, pl.ds(16 * j, 16)] = jnp.zeros((16,), jnp.float32)
            return carry

        lax.fori_loop(0, CH, zero_row, 0)
        abase = sid * RPT
        for k in range(RPT // CH):
            pltpu.sync_copy(r2_v, acc.at[pl.ds(abase + k * CH, CH)])
        pltpu.sync_copy(packed_hbm.at[pl.ds(cbase, 4)], pa_v)
        pltpu.sync_copy(w_hbm.at[pl.ds(cbase, 4)], wa_v)
        plsc.subcore_barrier()

        def subiter(c0, pbuf, wbuf, pnext, wnext):
            cf = jnp.minimum(c0 + 4, nck - 4)
            fp = pltpu.async_copy(packed_hbm.at[pl.ds(cbase + cf, 4)], pnext, fsm)
            fw = pltpu.async_copy(w_hbm.at[pl.ds(cbase + cf, 4)], wnext, fsm)
            gd = [pltpu.async_copy(x_hbm.at[pbuf.at[i, 0]], rows[i], gsem[i])
                  for i in range(4)]
            sd = []
            for i in range(4):
                gd[i].wait()
                for g in range(CH // 16):
                    didx[i][pl.ds(16 * g, 16)] = pbuf[i, 1, pl.ds(16 * g, 16)]
                scale(i, pbuf, wbuf)
                sd.append(pltpu.async_copy(rows[i], acc.at[didx[i]], ssem[i],
                                           add=True))
            for d in sd:
                d.wait()
            fp.wait()
            fw.wait()

        def body(s2, carry):
            c0 = 8 * s2
            subiter(c0, pa_v, wa_v, pb_v, wb_v)
            subiter(c0 + 4, pb_v, wb_v, pa_v, wa_v)
            return carry

        lax.fori_loop(0, nck // 8, body, 0)
        plsc.subcore_barrier()

        pltpu.sync_copy(acc.at[pl.ds(abase, RPT)],
                        part_hbm.at[pl.ds(cid * NP + abase, RPT)])


@jax.jit
def _sc_layer(x, packed, w):
    mesh = plsc.VectorSubcoreMesh(core_axis_name="c", subcore_axis_name="s")
    return pl.kernel(
        _sc_layer_body,
        out_type=jax.ShapeDtypeStruct((NC * NP, D), jnp.float32),
        mesh=mesh,
        scratch_types=(
            [pltpu.VMEM((CH, D), jnp.float32)] * 4
            + [pltpu.VMEM((4, 2, CH), jnp.int32)] * 2
            + [pltpu.VMEM((4, CH), jnp.float32)] * 2
            + [pltpu.VMEM((CH,), jnp.int32)] * 4
            + [pltpu.VMEM_SHARED((NP, D), jnp.float32)]
            + [pltpu.SemaphoreType.DMA] * 9
        ),
    )(x, packed, w)


def _mean_body(p0_ref, p1_ref, x0_ref, x1_ref, x2_ref, m_ref):
    m_ref[...] = (x0_ref[...] + x1_ref[...] + x2_ref[...]
                  + p0_ref[...] + p1_ref[...]) * 0.25


def _add_body(p0_ref, p1_ref, x_ref):
    x_ref[...] = p0_ref[...] + p1_ref[...]


@jax.jit
def _add2(p0, p1):
    return pl.pallas_call(
        _add_body,
        grid=(NP // _BLK2,),
        in_specs=[_row_spec2(), _row_spec2()],
        out_specs=_row_spec2(),
        out_shape=jax.ShapeDtypeStruct((NP, D), jnp.float32),
    )(p0, p1)


_BLK = 1280
_BLK2 = 1280


def _row_spec():
    return pl.BlockSpec((_BLK, D), lambda i: (i, 0))


def _row_spec2():
    return pl.BlockSpec((_BLK2, D), lambda i: (i, 0))


@jax.jit
def _mean5(p0, p1, x0, x1, x2):
    return pl.pallas_call(
        _mean_body,
        grid=(NP // _BLK,),
        in_specs=[_row_spec()] * 5,
        out_specs=_row_spec(),
        out_shape=jax.ShapeDtypeStruct((NP, D), jnp.float32),
    )(p0, p1, x0, x1, x2)


def kernel(user_emb, item_emb, edge_weight, edge_index):
    x0 = jnp.pad(jnp.concatenate([user_emb, item_emb], axis=0),
                 ((0, NP - NN), (0, 0)))
    pad = EPAD - NE
    src = jnp.pad(edge_index[1], (0, pad)).reshape(-1, CH)
    dst = jnp.pad(edge_index[0], (0, pad)).reshape(-1, CH)
    w = jnp.pad(edge_weight, (0, pad)).reshape(-1, CH)
    packed = jnp.stack([src, dst], axis=1)

    xs = [x0]
    for layer in range(NLAYER - 1):
        part = _sc_layer(xs[-1], packed, w)
        xs.append(_add2(part[:NP], part[NP:]))
    part = _sc_layer(xs[-1], packed, w)
    mean = _mean5(part[:NP], part[NP:], *xs)
    return (mean[:NU], mean[NU:NN])
